# baseline (speedup 1.0000x reference)
import functools

import jax
import jax.numpy as jnp
from jax import lax
from jax.experimental import pallas as pl
from jax.experimental.pallas import tpu as pltpu
from jax.experimental.pallas import tpu_sc as plsc

B, S, D = 16384, 200, 32
N = B * S

NH, ND, NM = 24, 7, 12
CT_ROWS = NH * ND * NM
TCAT = 48
CT_PK = CT_ROWS // 4

_info = plsc.get_sparse_core_info()
NC, NS, L = _info.num_cores, _info.num_subcores, _info.num_lanes
NW = NC * NS
B_PER_W = B // NW
OC = 4
NOC = B_PER_W // OC
IDXW = OC * S
GFULL = S // L


def _ct_body(tcat_ref, ct_ref):
    blocks = []
    for j in range(4):
        rr = lax.broadcasted_iota(jnp.int32, (CT_PK, TCAT), 0) * 4 + j
        cc = lax.broadcasted_iota(jnp.int32, (CT_PK, TCAT), 1)
        h = rr // (ND * NM)
        rem = rr - h * (ND * NM)
        d = rem // NM
        m = rem - d * NM
        mh = ((cc == h) | (cc == NH + d) | (cc == NH + ND + m)).astype(jnp.float32)
        blocks.append(lax.dot_general(
            mh, tcat_ref[...], (((1,), (0,)), ((), ())),
            preferred_element_type=jnp.float32,
            precision=lax.Precision.HIGHEST))
    ct_ref[...] = lax.concatenate(blocks, 1)


def _build_combined_table(hour_table, day_table, month_table):
    tcat = jnp.concatenate(
        [hour_table, day_table, month_table,
         jnp.zeros((TCAT - NH - ND - NM, D), jnp.float32)], axis=0)
    return pl.pallas_call(
        _ct_body,
        out_shape=jax.ShapeDtypeStruct((CT_PK, 128), jnp.float32),
    )(tcat)


def _make_sc_body(bp_per_w, noc):
  def _sc_body(ct_hbm, h_hbm, d_hbm, m_hbm, out_hbm,
             ct_v, h0_v, h1_v, d0_v, d1_v, m0_v, m1_v,
             stage_a, stage_b, sem_l, sem_s):
    B_PER_W, NOC = bp_per_w, noc
    wid = lax.axis_index("s") * NC + lax.axis_index("c")
    base = wid * (B_PER_W * S)
    bbase = wid * B_PER_W
    pltpu.sync_copy(ct_hbm, ct_v)
    lane = lax.broadcasted_iota(jnp.int32, (L,), 0)
    idx_bufs = ((h0_v, d0_v, m0_v), (h1_v, d1_v, m1_v))
    stages = (stage_a, stage_b)

    def fire_idx(oi, par):
        off = base + oi * IDXW
        for src, dst in zip((h_hbm, d_hbm, m_hbm), idx_bufs[par]):
            pltpu.async_copy(src.at[pl.ds(off, IDXW)],
                             dst.at[pl.ds(0, IDXW)], sem_l)

    def wait_idx(par):
        for dst in idx_bufs[par]:
            pltpu.make_async_copy(h_hbm.at[pl.ds(0, IDXW)],
                                  dst.at[pl.ds(0, IDXW)], sem_l).wait()

    def wait_store(slot):
        pltpu.make_async_copy(stages[slot], out_hbm.at[pl.ds(bbase, 1)],
                              sem_s).wait()

    def do_group(par, o, p, slot, mask):
        hb, db, mb = idx_bufs[par]
        hv = hb[pl.ds(o, L)]
        dv = db[pl.ds(o, L)]
        mv = mb[pl.ds(o, L)]
        c = hv * (ND * NM) + dv * NM + mv
        cs = c << 5
        sv = p + lane
        st2 = stages[slot].at[0]

        def col_body(j, c3):
            ca = (lane + j) & (D - 1)
            cb = ca ^ (D // 2)
            va = plsc.load_gather(ct_v, [cs | ca], mask=mask)
            vb = plsc.load_gather(ct_v, [cs | cb], mask=mask)
            plsc.store_scatter(st2, [sv, ca], va, mask=mask)
            plsc.store_scatter(st2, [sv, cb], vb, mask=mask)
            return c3

        lax.fori_loop(0, 1, col_body, 0, unroll=1)

    def half(t, oi, par):
        wait_idx(par)

        @pl.when(oi < NOC - 1)
        def _():
            fire_idx(oi + 1, 1 - par)

        for bi in range(OC):
            slot = bi & 1
            if bi < 2:
                @pl.when(oi >= 1)
                def _():
                    wait_store(slot)
            else:
                wait_store(slot)

            def grp(g, c2, _bi=bi, _slot=slot, _par=par):
                do_group(_par, _bi * S + g * L, g * L, _slot, None)
                return c2

            lax.fori_loop(0, GFULL, grp, 0)
            do_group(par, bi * S + GFULL * L, GFULL * L, slot,
                     lane < S - GFULL * L)

            pltpu.async_copy(stages[slot],
                             out_hbm.at[pl.ds(bbase + oi * OC + bi, 1)],
                             sem_s)

    fire_idx(0, 0)

    def outer(t, carry):
        half(t, 2 * t, 0)
        half(t, 2 * t + 1, 1)
        return carry

    lax.fori_loop(0, NOC // 2, outer, 0)
    wait_store(0)
    wait_store(1)
  return _sc_body


def _make_sc_call(bp):
  bp_per_w = bp // NW
  return functools.partial(
    pl.kernel,
    out_type=jax.ShapeDtypeStruct((bp, S, D), jnp.float32),
    mesh=plsc.VectorSubcoreMesh(core_axis_name="c", subcore_axis_name="s"),
    scratch_types=[
        pltpu.VMEM((CT_PK * 128,), jnp.float32),
        pltpu.VMEM((IDXW + 2 * L,), jnp.int32),
        pltpu.VMEM((IDXW + 2 * L,), jnp.int32),
        pltpu.VMEM((IDXW + 2 * L,), jnp.int32),
        pltpu.VMEM((IDXW + 2 * L,), jnp.int32),
        pltpu.VMEM((IDXW + 2 * L,), jnp.int32),
        pltpu.VMEM((IDXW + 2 * L,), jnp.int32),
        pltpu.VMEM((1, S, D), jnp.float32),
        pltpu.VMEM((1, S, D), jnp.float32),
        pltpu.SemaphoreType.DMA,
        pltpu.SemaphoreType.DMA,
    ],
    compiler_params=pltpu.CompilerParams(use_tc_tiling_on_sc=True,
                                         needs_layout_passes=False),
  )(_make_sc_body(bp_per_w, bp_per_w // OC))


_sc_call = _make_sc_call(B)


def kernel(hours, days, months, hour_table, day_table, month_table):
    h = hours.astype(jnp.int32).reshape(N)
    d = days.astype(jnp.int32).reshape(N)
    m = months.astype(jnp.int32).reshape(N)
    ct = _build_combined_table(hour_table, day_table, month_table).reshape(CT_PK * 128)
    return _sc_call(ct, h, d, m)


# speedup vs baseline: 1.6122x; 1.6122x over previous
import functools

import jax
import jax.numpy as jnp
from jax import lax
from jax.experimental import pallas as pl
from jax.experimental.pallas import tpu as pltpu
from jax.experimental.pallas import tpu_sc as plsc

B, S, D = 16384, 200, 32
N = B * S

NH, ND, NM = 24, 7, 12
CT_ROWS = NH * ND * NM
TCAT = 48
CT_PK = CT_ROWS // 4

_info = plsc.get_sparse_core_info()
NC, NS, L = _info.num_cores, _info.num_subcores, _info.num_lanes
NW = NC * NS
B_PER_W = B // NW
OC = 4
NOC = B_PER_W // OC
IDXW = OC * S
GFULL = S // L


def _ct_body(tcat_ref, ct_ref):
    blocks = []
    for j in range(4):
        rr = lax.broadcasted_iota(jnp.int32, (CT_PK, TCAT), 0) * 4 + j
        cc = lax.broadcasted_iota(jnp.int32, (CT_PK, TCAT), 1)
        h = rr // (ND * NM)
        rem = rr - h * (ND * NM)
        d = rem // NM
        m = rem - d * NM
        mh = ((cc == h) | (cc == NH + d) | (cc == NH + ND + m)).astype(jnp.float32)
        blocks.append(lax.dot_general(
            mh, tcat_ref[...], (((1,), (0,)), ((), ())),
            preferred_element_type=jnp.float32,
            precision=lax.Precision.HIGHEST))
    ct_ref[...] = lax.concatenate(blocks, 1)


def _build_combined_table(hour_table, day_table, month_table):
    tcat = jnp.concatenate(
        [hour_table, day_table, month_table,
         jnp.zeros((TCAT - NH - ND - NM, D), jnp.float32)], axis=0)
    return pl.pallas_call(
        _ct_body,
        out_shape=jax.ShapeDtypeStruct((CT_PK, 128), jnp.float32),
    )(tcat)


def _make_sc_body(bp_per_w, noc):
  def _sc_body(ct_hbm, h_hbm, d_hbm, m_hbm, out_hbm,
             ct_v, h0_v, h1_v, d0_v, d1_v, m0_v, m1_v,
             stage_a, stage_b, sem_l, sem_s):
    B_PER_W, NOC = bp_per_w, noc
    wid = lax.axis_index("s") * NC + lax.axis_index("c")
    base = wid * (B_PER_W * S)
    bbase = wid * B_PER_W
    pltpu.sync_copy(ct_hbm, ct_v)
    lane = lax.broadcasted_iota(jnp.int32, (L,), 0)
    idx_bufs = ((h0_v, d0_v, m0_v), (h1_v, d1_v, m1_v))
    stages = (stage_a, stage_b)

    def fire_idx(oi, par):
        off = base + oi * IDXW
        for src, dst in zip((h_hbm, d_hbm, m_hbm), idx_bufs[par]):
            pltpu.async_copy(src.at[pl.ds(off, IDXW)],
                             dst.at[pl.ds(0, IDXW)], sem_l)

    def wait_idx(par):
        for dst in idx_bufs[par]:
            pltpu.make_async_copy(h_hbm.at[pl.ds(0, IDXW)],
                                  dst.at[pl.ds(0, IDXW)], sem_l).wait()

    def wait_store(slot):
        pltpu.make_async_copy(stages[slot], out_hbm.at[pl.ds(bbase, 1)],
                              sem_s).wait()

    def do_group(par, o, p, slot, mask):
        hb, db, mb = idx_bufs[par]
        hv = hb[pl.ds(o, L)]
        dv = db[pl.ds(o, L)]
        mv = mb[pl.ds(o, L)]
        c = hv * (ND * NM) + dv * NM + mv
        cs = c << 5
        sb = (p + lane) << 5
        st2 = stages[slot].at[0]

        def col_body(j, c3):
            ca = (lane + j) & (D - 1)
            cb = ca ^ (D // 2)
            va = plsc.load_gather(ct_v, [cs | ca], mask=mask)
            vb = plsc.load_gather(ct_v, [cs | cb], mask=mask)
            fa = sb | ca
            fb = sb | cb
            plsc.store_scatter(st2, [fa >> 7, fa & 127], va, mask=mask)
            plsc.store_scatter(st2, [fb >> 7, fb & 127], vb, mask=mask)
            return c3

        lax.fori_loop(0, D // 2, col_body, 0, unroll=8)

    def half(t, oi, par):
        wait_idx(par)

        @pl.when(oi < NOC - 1)
        def _():
            fire_idx(oi + 1, 1 - par)

        for bi in range(OC):
            slot = bi & 1
            if bi < 2:
                @pl.when(oi >= 1)
                def _():
                    wait_store(slot)
            else:
                wait_store(slot)

            def grp(g, c2, _bi=bi, _slot=slot, _par=par):
                do_group(_par, _bi * S + g * L, g * L, _slot, None)
                return c2

            lax.fori_loop(0, GFULL, grp, 0)
            do_group(par, bi * S + GFULL * L, GFULL * L, slot,
                     lane < S - GFULL * L)

            pltpu.async_copy(stages[slot],
                             out_hbm.at[pl.ds(bbase + oi * OC + bi, 1)],
                             sem_s)

    fire_idx(0, 0)

    def outer(t, carry):
        half(t, 2 * t, 0)
        half(t, 2 * t + 1, 1)
        return carry

    lax.fori_loop(0, NOC // 2, outer, 0)
    wait_store(0)
    wait_store(1)
  return _sc_body


def _make_sc_call(bp):
  bp_per_w = bp // NW
  return functools.partial(
    pl.kernel,
    out_type=jax.ShapeDtypeStruct((bp, S * D // 128, 128), jnp.float32),
    mesh=plsc.VectorSubcoreMesh(core_axis_name="c", subcore_axis_name="s"),
    scratch_types=[
        pltpu.VMEM((CT_PK * 128,), jnp.float32),
        pltpu.VMEM((IDXW + 2 * L,), jnp.int32),
        pltpu.VMEM((IDXW + 2 * L,), jnp.int32),
        pltpu.VMEM((IDXW + 2 * L,), jnp.int32),
        pltpu.VMEM((IDXW + 2 * L,), jnp.int32),
        pltpu.VMEM((IDXW + 2 * L,), jnp.int32),
        pltpu.VMEM((IDXW + 2 * L,), jnp.int32),
        pltpu.VMEM((1, S * D // 128, 128), jnp.float32),
        pltpu.VMEM((1, S * D // 128, 128), jnp.float32),
        pltpu.SemaphoreType.DMA,
        pltpu.SemaphoreType.DMA,
    ],
    compiler_params=pltpu.CompilerParams(use_tc_tiling_on_sc=True,
                                         needs_layout_passes=False),
  )(_make_sc_body(bp_per_w, bp_per_w // OC))


_sc_call = _make_sc_call(B)


def kernel(hours, days, months, hour_table, day_table, month_table):
    h = hours.astype(jnp.int32).reshape(N)
    d = days.astype(jnp.int32).reshape(N)
    m = months.astype(jnp.int32).reshape(N)
    ct = _build_combined_table(hour_table, day_table, month_table).reshape(CT_PK * 128)
    return _sc_call(ct, h, d, m).reshape(B, S, D)
